# baseline (speedup 1.0000x reference)
import functools

import jax
import jax.numpy as jnp
from jax import lax
from jax.experimental import pallas as pl
from jax.experimental.pallas import tpu as pltpu
from jax.experimental.pallas import tpu_sc as plsc

B = 16384
U = 100000
D = 64
DV, DA, DT = 512, 128, 768

NC, NS = 2, 16
NW = NC * NS
H = B // 2
BPW = H // NW
MCHUNK = 32
NCH = BPW // MCHUNK
UCHUNK = 128

_sc_mesh = plsc.VectorSubcoreMesh(core_axis_name="c", subcore_axis_name="s")
_sc_params = pltpu.CompilerParams(use_tc_tiling_on_sc=True)


def _make_mgather(off):
    def body(mid_hbm, vf_hbm, af_hbm, tf_hbm,
             vout, aout, tout,
             midx, vb0, ab0, tb0, vb1, ab1, tb1, gs0, gs1, ss0, ss1):
        wid = lax.axis_index("s") * NC + lax.axis_index("c")
        base = wid * BPW
        pltpu.sync_copy(mid_hbm.at[pl.ds(off + base, BPW)], midx)
        bufs = ((vb0, ab0, tb0, gs0, ss0), (vb1, ab1, tb1, gs1, ss1))

        def fire_gather(k, c):
            vb, ab, tb, gs, _ = bufs[k]
            o = c * MCHUNK
            idx = midx.at[pl.ds(o, MCHUNK)]
            return [pltpu.async_copy(vf_hbm.at[idx], vb, gs),
                    pltpu.async_copy(af_hbm.at[idx], ab, gs),
                    pltpu.async_copy(tf_hbm.at[idx], tb, gs)]

        def fire_scatter(k, c):
            vb, ab, tb, _, ss = bufs[k]
            o = base + c * MCHUNK
            return [pltpu.async_copy(vb, vout.at[pl.ds(o, MCHUNK)], ss),
                    pltpu.async_copy(ab, aout.at[pl.ds(o, MCHUNK)], ss),
                    pltpu.async_copy(tb, tout.at[pl.ds(o, MCHUNK)], ss)]

        gh = [None, None]
        sh = [None, None]
        gh[0] = fire_gather(0, 0)
        for c in range(NCH):
            k = c & 1
            for h in gh[k]:
                h.wait()
            if c + 1 < NCH:
                nk = (c + 1) & 1
                if sh[nk] is not None:
                    for h in sh[nk]:
                        h.wait()
                gh[nk] = fire_gather(nk, c + 1)
            sh[k] = fire_scatter(k, c)
        for h in sh[0]:
            h.wait()
        for h in sh[1]:
            h.wait()

    return pl.kernel(
        body,
        out_type=[
            jax.ShapeDtypeStruct((H, DV), jnp.float32),
            jax.ShapeDtypeStruct((H, DA), jnp.float32),
            jax.ShapeDtypeStruct((H, DT), jnp.float32),
        ],
        mesh=_sc_mesh,
        compiler_params=_sc_params,
        scratch_types=[
            pltpu.VMEM((BPW,), jnp.int32),
            pltpu.VMEM((MCHUNK, DV), jnp.float32),
            pltpu.VMEM((MCHUNK, DA), jnp.float32),
            pltpu.VMEM((MCHUNK, DT), jnp.float32),
            pltpu.VMEM((MCHUNK, DV), jnp.float32),
            pltpu.VMEM((MCHUNK, DA), jnp.float32),
            pltpu.VMEM((MCHUNK, DT), jnp.float32),
            pltpu.SemaphoreType.DMA,
            pltpu.SemaphoreType.DMA,
            pltpu.SemaphoreType.DMA,
            pltpu.SemaphoreType.DMA,
        ],
    )


def _make_ugather(off):
    def body(uid_hbm, ut_hbm, uout, uidx, ub0, ub1, s0, s1):
        wid = lax.axis_index("s") * NC + lax.axis_index("c")
        base = wid * BPW
        pltpu.sync_copy(uid_hbm.at[pl.ds(off + base, BPW)], uidx)
        g0 = pltpu.async_copy(ut_hbm.at[uidx.at[pl.ds(0, UCHUNK)]], ub0, s0)
        g1 = pltpu.async_copy(ut_hbm.at[uidx.at[pl.ds(UCHUNK, UCHUNK)]], ub1, s1)
        g0.wait()
        w0 = pltpu.async_copy(ub0, uout.at[pl.ds(base, UCHUNK)], s0)
        g1.wait()
        w1 = pltpu.async_copy(ub1, uout.at[pl.ds(base + UCHUNK, UCHUNK)], s1)
        w0.wait()
        w1.wait()

    return pl.kernel(
        body,
        out_type=jax.ShapeDtypeStruct((H, 2 * D), jnp.float32),
        mesh=_sc_mesh,
        compiler_params=_sc_params,
        scratch_types=[
            pltpu.VMEM((BPW,), jnp.int32),
            pltpu.VMEM((UCHUNK, 2 * D), jnp.float32),
            pltpu.VMEM((UCHUNK, 2 * D), jnp.float32),
            pltpu.SemaphoreType.DMA,
            pltpu.SemaphoreType.DMA,
        ],
    )


_mg0 = _make_mgather(0)
_mg1 = _make_mgather(H)
_ug0 = _make_ugather(0)
_ug1 = _make_ugather(H)


PADR = 4000


def _pad_body(u_ref, o_ref):
    o_ref[:, :D] = u_ref[...]
    o_ref[:, D:] = jnp.zeros((PADR, D), jnp.float32)


def _pad_table(user_table):
    return pl.pallas_call(
        _pad_body,
        grid=(U // PADR,),
        in_specs=[pl.BlockSpec((PADR, D), lambda i: (i, 0))],
        out_specs=pl.BlockSpec((PADR, 2 * D), lambda i: (i, 0)),
        out_shape=jax.ShapeDtypeStruct((U, 2 * D), jnp.float32),
    )(user_table)


BT = 1024


def _fuse_body(u_ref, v_ref, a_ref, t_ref, wv_ref, wa_ref, wt_ref,
               bmm_ref, wout_ref, bout_ref, o_ref):
    memb = jnp.dot(v_ref[...], wv_ref[...], preferred_element_type=jnp.float32)
    memb += jnp.dot(a_ref[...], wa_ref[...], preferred_element_type=jnp.float32)
    memb += jnp.dot(t_ref[...], wt_ref[...], preferred_element_type=jnp.float32)
    memb += bmm_ref[...]
    mu = jnp.sum(memb * u_ref[:, :D], axis=1)
    o_ref[...] = jax.nn.sigmoid(mu * wout_ref[0, 0] + bout_ref[0, 0])


def _fuse(uemb, mv, ma, mt, Wv, Wa, Wt, bmm, wout, bout):
    return pl.pallas_call(
        _fuse_body,
        grid=(H // BT,),
        in_specs=[
            pl.BlockSpec((BT, 2 * D), lambda i: (i, 0)),
            pl.BlockSpec((BT, DV), lambda i: (i, 0)),
            pl.BlockSpec((BT, DA), lambda i: (i, 0)),
            pl.BlockSpec((BT, DT), lambda i: (i, 0)),
            pl.BlockSpec((DV, D), lambda i: (0, 0)),
            pl.BlockSpec((DA, D), lambda i: (0, 0)),
            pl.BlockSpec((DT, D), lambda i: (0, 0)),
            pl.BlockSpec((1, D), lambda i: (0, 0)),
            pl.BlockSpec((1, 1), lambda i: (0, 0)),
            pl.BlockSpec((1, 1), lambda i: (0, 0)),
        ],
        out_specs=pl.BlockSpec((BT,), lambda i: (i,)),
        out_shape=jax.ShapeDtypeStruct((H,), jnp.float32),
    )(uemb, mv, ma, mt, Wv, Wa, Wt, bmm, wout, bout)


def kernel(x, user_table, video_feat, audio_feat, text_feat, W_mm, b_mm, W_out, b_out):
    x = x.astype(jnp.int32)
    uid, mid = jax.lax.optimization_barrier((x[0], x[1]))
    ut_pad = _pad_table(user_table)
    Wv = W_mm[:DV]
    Wa = W_mm[DV:DV + DA]
    Wt = W_mm[DV + DA:]
    bmm = b_mm.reshape(1, D)
    bout = b_out.reshape(1, 1)

    mv0, ma0, mt0 = _mg0(mid, video_feat, audio_feat, text_feat)
    ue0 = _ug0(uid, ut_pad)
    mv1, ma1, mt1 = _mg1(mid, video_feat, audio_feat, text_feat)
    ue1 = _ug1(uid, ut_pad)

    o0 = _fuse(ue0, mv0, ma0, mt0, Wv, Wa, Wt, bmm, W_out, bout)
    o1 = _fuse(ue1, mv1, ma1, mt1, Wv, Wa, Wt, bmm, W_out, bout)
    return jnp.concatenate([o0, o1]).reshape(B, 1)


# speedup vs baseline: 1.1155x; 1.1155x over previous
import functools

import jax
import jax.numpy as jnp
from jax import lax
from jax.experimental import pallas as pl
from jax.experimental.pallas import tpu as pltpu
from jax.experimental.pallas import tpu_sc as plsc

B = 16384
U = 100000
D = 64
DV, DA, DT = 512, 128, 768

NC, NS = 2, 16
NW = NC * NS
H = B // 2
BPW = H // NW
MCHUNK = 32
NCH = BPW // MCHUNK
UCHUNK = 128

_sc_mesh = plsc.VectorSubcoreMesh(core_axis_name="c", subcore_axis_name="s")
_sc_params = pltpu.CompilerParams(use_tc_tiling_on_sc=True)


def _make_mgather(off):
    def body(mid_hbm, vf_hbm, af_hbm, tf_hbm,
             vout, aout, tout,
             midx, vb0, ab0, tb0, vb1, ab1, tb1, gs0, gs1, ss0, ss1):
        wid = lax.axis_index("s") * NC + lax.axis_index("c")
        base = wid * BPW
        pltpu.sync_copy(mid_hbm.at[pl.ds(off + base, BPW)], midx)
        bufs = ((vb0, ab0, tb0, gs0, ss0), (vb1, ab1, tb1, gs1, ss1))

        def fire_gather(k, c):
            vb, ab, tb, gs, _ = bufs[k]
            o = c * MCHUNK
            idx = midx.at[pl.ds(o, MCHUNK)]
            return [pltpu.async_copy(vf_hbm.at[idx], vb, gs),
                    pltpu.async_copy(af_hbm.at[idx], ab, gs),
                    pltpu.async_copy(tf_hbm.at[idx], tb, gs)]

        def fire_scatter(k, c):
            vb, ab, tb, _, ss = bufs[k]
            o = base + c * MCHUNK
            return [pltpu.async_copy(vb, vout.at[pl.ds(o, MCHUNK)], ss),
                    pltpu.async_copy(ab, aout.at[pl.ds(o, MCHUNK)], ss),
                    pltpu.async_copy(tb, tout.at[pl.ds(o, MCHUNK)], ss)]

        gh = [None, None]
        sh = [None, None]
        gh[0] = fire_gather(0, 0)
        for c in range(NCH):
            k = c & 1
            for h in gh[k]:
                h.wait()
            if c + 1 < NCH:
                nk = (c + 1) & 1
                if sh[nk] is not None:
                    for h in sh[nk]:
                        h.wait()
                gh[nk] = fire_gather(nk, c + 1)
            sh[k] = fire_scatter(k, c)
        for h in sh[0]:
            h.wait()
        for h in sh[1]:
            h.wait()

    return pl.kernel(
        body,
        out_type=[
            jax.ShapeDtypeStruct((H, DV), jnp.float32),
            jax.ShapeDtypeStruct((H, DA), jnp.float32),
            jax.ShapeDtypeStruct((H, DT), jnp.float32),
        ],
        mesh=_sc_mesh,
        compiler_params=_sc_params,
        scratch_types=[
            pltpu.VMEM((BPW,), jnp.int32),
            pltpu.VMEM((MCHUNK, DV), jnp.float32),
            pltpu.VMEM((MCHUNK, DA), jnp.float32),
            pltpu.VMEM((MCHUNK, DT), jnp.float32),
            pltpu.VMEM((MCHUNK, DV), jnp.float32),
            pltpu.VMEM((MCHUNK, DA), jnp.float32),
            pltpu.VMEM((MCHUNK, DT), jnp.float32),
            pltpu.SemaphoreType.DMA,
            pltpu.SemaphoreType.DMA,
            pltpu.SemaphoreType.DMA,
            pltpu.SemaphoreType.DMA,
        ],
    )


def _make_ugather(off):
    def body(uid_hbm, ut_hbm, uout, uidx, ub0, ub1, s0, s1):
        wid = lax.axis_index("s") * NC + lax.axis_index("c")
        base = wid * BPW
        pltpu.sync_copy(uid_hbm.at[pl.ds(off + base, BPW)], uidx)
        g0 = pltpu.async_copy(ut_hbm.at[uidx.at[pl.ds(0, UCHUNK)]], ub0, s0)
        g1 = pltpu.async_copy(ut_hbm.at[uidx.at[pl.ds(UCHUNK, UCHUNK)]], ub1, s1)
        g0.wait()
        w0 = pltpu.async_copy(ub0, uout.at[pl.ds(base, UCHUNK)], s0)
        g1.wait()
        w1 = pltpu.async_copy(ub1, uout.at[pl.ds(base + UCHUNK, UCHUNK)], s1)
        w0.wait()
        w1.wait()

    return pl.kernel(
        body,
        out_type=jax.ShapeDtypeStruct((H, 2 * D), jnp.float32),
        mesh=_sc_mesh,
        compiler_params=_sc_params,
        scratch_types=[
            pltpu.VMEM((BPW,), jnp.int32),
            pltpu.VMEM((UCHUNK, 2 * D), jnp.float32),
            pltpu.VMEM((UCHUNK, 2 * D), jnp.float32),
            pltpu.SemaphoreType.DMA,
            pltpu.SemaphoreType.DMA,
        ],
    )


_mg0 = _make_mgather(0)
_mg1 = _make_mgather(H)
_ug0 = _make_ugather(0)
_ug1 = _make_ugather(H)


BT = 2048


def _fuse_body(u_ref, v_ref, a_ref, t_ref, wv_ref, wa_ref, wt_ref,
               bmm_ref, wout_ref, bout_ref, o_ref):
    memb = jnp.dot(v_ref[...], wv_ref[...], preferred_element_type=jnp.float32)
    memb += jnp.dot(a_ref[...], wa_ref[...], preferred_element_type=jnp.float32)
    memb += jnp.dot(t_ref[...], wt_ref[...], preferred_element_type=jnp.float32)
    memb += bmm_ref[...]
    mu = jnp.sum(memb * u_ref[:, :D], axis=1)
    o_ref[...] = jax.nn.sigmoid(mu * wout_ref[0, 0] + bout_ref[0, 0])


def _fuse(uemb, mv, ma, mt, Wv, Wa, Wt, bmm, wout, bout):
    return pl.pallas_call(
        _fuse_body,
        grid=(H // BT,),
        in_specs=[
            pl.BlockSpec((BT, 2 * D), lambda i: (i, 0)),
            pl.BlockSpec((BT, DV), lambda i: (i, 0)),
            pl.BlockSpec((BT, DA), lambda i: (i, 0)),
            pl.BlockSpec((BT, DT), lambda i: (i, 0)),
            pl.BlockSpec((DV, D), lambda i: (0, 0)),
            pl.BlockSpec((DA, D), lambda i: (0, 0)),
            pl.BlockSpec((DT, D), lambda i: (0, 0)),
            pl.BlockSpec((1, D), lambda i: (0, 0)),
            pl.BlockSpec((1, 1), lambda i: (0, 0)),
            pl.BlockSpec((1, 1), lambda i: (0, 0)),
        ],
        out_specs=pl.BlockSpec((BT,), lambda i: (i,)),
        out_shape=jax.ShapeDtypeStruct((H,), jnp.float32),
    )(uemb, mv, ma, mt, Wv, Wa, Wt, bmm, wout, bout)


def kernel(x, user_table, video_feat, audio_feat, text_feat, W_mm, b_mm, W_out, b_out):
    x = x.astype(jnp.int32)
    uid, mid = jax.lax.optimization_barrier((x[0], x[1]))
    ut_pad = jnp.pad(user_table, ((0, 0), (0, D)))
    Wv = W_mm[:DV]
    Wa = W_mm[DV:DV + DA]
    Wt = W_mm[DV + DA:]
    bmm = b_mm.reshape(1, D)
    bout = b_out.reshape(1, 1)

    mv0, ma0, mt0 = _mg0(mid, video_feat, audio_feat, text_feat)
    ue0 = _ug0(uid, ut_pad)
    mv1, ma1, mt1 = _mg1(mid, video_feat, audio_feat, text_feat)
    ue1 = _ug1(uid, ut_pad)

    o0 = _fuse(ue0, mv0, ma0, mt0, Wv, Wa, Wt, bmm, W_out, bout)
    o1 = _fuse(ue1, mv1, ma1, mt1, Wv, Wa, Wt, bmm, W_out, bout)
    return jnp.concatenate([o0, o1]).reshape(B, 1)


# speedup vs baseline: 1.1167x; 1.0011x over previous
import jax
import jax.numpy as jnp
from jax import lax
from jax.experimental import pallas as pl
from jax.experimental.pallas import tpu as pltpu
from jax.experimental.pallas import tpu_sc as plsc

B = 16384
U = 100000
D = 64
DV, DA, DT = 512, 128, 768

NC, NS = 2, 16
NW = NC * NS
H = B // 2
BPW = H // NW
MCHUNK = 32
NCH = BPW // MCHUNK
UCHUNK = 128

_sc_mesh = plsc.VectorSubcoreMesh(core_axis_name="c", subcore_axis_name="s")
_sc_params = pltpu.CompilerParams(use_tc_tiling_on_sc=True)


def _make_mgather(off):
    def body(mid_hbm, vf_hbm, af_hbm, tf_hbm,
             vout, aout, tout,
             midx, vb0, ab0, tb0, vb1, ab1, tb1, gs0, gs1, ss0, ss1):
        wid = lax.axis_index("s") * NC + lax.axis_index("c")
        base = wid * BPW
        pltpu.sync_copy(mid_hbm.at[pl.ds(off + base, BPW)], midx)
        bufs = ((vb0, ab0, tb0, gs0, ss0), (vb1, ab1, tb1, gs1, ss1))

        def fire_gather(k, c):
            vb, ab, tb, gs, _ = bufs[k]
            o = c * MCHUNK
            idx = midx.at[pl.ds(o, MCHUNK)]
            return [pltpu.async_copy(vf_hbm.at[idx], vb, gs),
                    pltpu.async_copy(af_hbm.at[idx], ab, gs),
                    pltpu.async_copy(tf_hbm.at[idx], tb, gs)]

        def fire_scatter(k, c):
            vb, ab, tb, _, ss = bufs[k]
            o = base + c * MCHUNK
            return [pltpu.async_copy(vb, vout.at[pl.ds(o, MCHUNK)], ss),
                    pltpu.async_copy(ab, aout.at[pl.ds(o, MCHUNK)], ss),
                    pltpu.async_copy(tb, tout.at[pl.ds(o, MCHUNK)], ss)]

        gh = [None, None]
        sh = [None, None]
        gh[0] = fire_gather(0, 0)
        for c in range(NCH):
            k = c & 1
            for h in gh[k]:
                h.wait()
            if c + 1 < NCH:
                nk = (c + 1) & 1
                if sh[nk] is not None:
                    for h in sh[nk]:
                        h.wait()
                gh[nk] = fire_gather(nk, c + 1)
            sh[k] = fire_scatter(k, c)
        for h in sh[0]:
            h.wait()
        for h in sh[1]:
            h.wait()

    return pl.kernel(
        body,
        out_type=[
            jax.ShapeDtypeStruct((H, DV), jnp.float32),
            jax.ShapeDtypeStruct((H, DA), jnp.float32),
            jax.ShapeDtypeStruct((H, DT), jnp.float32),
        ],
        mesh=_sc_mesh,
        compiler_params=_sc_params,
        scratch_types=[
            pltpu.VMEM((BPW,), jnp.int32),
            pltpu.VMEM((MCHUNK, DV), jnp.float32),
            pltpu.VMEM((MCHUNK, DA), jnp.float32),
            pltpu.VMEM((MCHUNK, DT), jnp.float32),
            pltpu.VMEM((MCHUNK, DV), jnp.float32),
            pltpu.VMEM((MCHUNK, DA), jnp.float32),
            pltpu.VMEM((MCHUNK, DT), jnp.float32),
            pltpu.SemaphoreType.DMA,
            pltpu.SemaphoreType.DMA,
            pltpu.SemaphoreType.DMA,
            pltpu.SemaphoreType.DMA,
        ],
    )


def _make_ugather(off):
    def body(uid_hbm, ut_hbm, uout, uidx, ub0, ub1, s0, s1):
        wid = lax.axis_index("s") * NC + lax.axis_index("c")
        base = wid * BPW
        pltpu.sync_copy(uid_hbm.at[pl.ds(off + base, BPW)], uidx)
        g0 = pltpu.async_copy(ut_hbm.at[uidx.at[pl.ds(0, UCHUNK)]], ub0, s0)
        g1 = pltpu.async_copy(ut_hbm.at[uidx.at[pl.ds(UCHUNK, UCHUNK)]], ub1, s1)
        g0.wait()
        w0 = pltpu.async_copy(ub0, uout.at[pl.ds(base, UCHUNK)], s0)
        g1.wait()
        w1 = pltpu.async_copy(ub1, uout.at[pl.ds(base + UCHUNK, UCHUNK)], s1)
        w0.wait()
        w1.wait()

    return pl.kernel(
        body,
        out_type=jax.ShapeDtypeStruct((H, 2 * D), jnp.float32),
        mesh=_sc_mesh,
        compiler_params=_sc_params,
        scratch_types=[
            pltpu.VMEM((BPW,), jnp.int32),
            pltpu.VMEM((UCHUNK, 2 * D), jnp.float32),
            pltpu.VMEM((UCHUNK, 2 * D), jnp.float32),
            pltpu.SemaphoreType.DMA,
            pltpu.SemaphoreType.DMA,
        ],
    )


_mg0 = _make_mgather(0)
_mg1 = _make_mgather(H)
_ug0 = _make_ugather(0)
_ug1 = _make_ugather(H)


BT = 1024


def _fuse_body(u_ref, v_ref, a_ref, t_ref, wv_ref, wa_ref, wt_ref,
               bmm_ref, wout_ref, bout_ref, o_ref):
    memb = jnp.dot(v_ref[...], wv_ref[...], preferred_element_type=jnp.float32)
    memb += jnp.dot(a_ref[...], wa_ref[...], preferred_element_type=jnp.float32)
    memb += jnp.dot(t_ref[...], wt_ref[...], preferred_element_type=jnp.float32)
    memb += bmm_ref[...]
    mu = jnp.sum(memb * u_ref[:, :D], axis=1)
    o_ref[...] = jax.nn.sigmoid(mu * wout_ref[0, 0] + bout_ref[0, 0])


def _fuse(uemb, mv, ma, mt, Wv, Wa, Wt, bmm, wout, bout):
    return pl.pallas_call(
        _fuse_body,
        grid=(H // BT,),
        in_specs=[
            pl.BlockSpec((BT, 2 * D), lambda i: (i, 0)),
            pl.BlockSpec((BT, DV), lambda i: (i, 0)),
            pl.BlockSpec((BT, DA), lambda i: (i, 0)),
            pl.BlockSpec((BT, DT), lambda i: (i, 0)),
            pl.BlockSpec((DV, D), lambda i: (0, 0)),
            pl.BlockSpec((DA, D), lambda i: (0, 0)),
            pl.BlockSpec((DT, D), lambda i: (0, 0)),
            pl.BlockSpec((1, D), lambda i: (0, 0)),
            pl.BlockSpec((1, 1), lambda i: (0, 0)),
            pl.BlockSpec((1, 1), lambda i: (0, 0)),
        ],
        out_specs=pl.BlockSpec((BT,), lambda i: (i,)),
        out_shape=jax.ShapeDtypeStruct((H,), jnp.float32),
    )(uemb, mv, ma, mt, Wv, Wa, Wt, bmm, wout, bout)


def kernel(x, user_table, video_feat, audio_feat, text_feat, W_mm, b_mm, W_out, b_out):
    x = x.astype(jnp.int32)
    uid, mid = jax.lax.optimization_barrier((x[0], x[1]))
    ut_pad = jnp.pad(user_table, ((0, 0), (0, D)))
    Wv = W_mm[:DV]
    Wa = W_mm[DV:DV + DA]
    Wt = W_mm[DV + DA:]
    bmm = b_mm.reshape(1, D)
    bout = b_out.reshape(1, 1)

    mv0, ma0, mt0 = _mg0(mid, video_feat, audio_feat, text_feat)
    ue0 = _ug0(uid, ut_pad)
    mv1, ma1, mt1 = _mg1(mid, video_feat, audio_feat, text_feat)
    ue1 = _ug1(uid, ut_pad)

    o0 = _fuse(ue0, mv0, ma0, mt0, Wv, Wa, Wt, bmm, W_out, bout)
    o1 = _fuse(ue1, mv1, ma1, mt1, Wv, Wa, Wt, bmm, W_out, bout)
    return jnp.concatenate([o0, o1]).reshape(B, 1)
